# baseline (speedup 1.0000x reference)
import jax
import jax.numpy as jnp
from jax import lax
from jax.experimental import pallas as pl
from jax.experimental.pallas import tpu as pltpu
from jax.experimental.pallas import tpu_sc as plsc

N = 10000
E = 320000
D = 128
DE = 16
L = 128
ENTRY = D + DE
SECOND = D + L

NC = 2
NS = 16
NW = NC * NS
NP = 10240
EP = NW * NP
CG = 80
GPW = NP // CG
RPT = NP // NS

_HI = lax.Precision.HIGHEST
_f32 = jnp.float32

_SC_MESH = plsc.VectorSubcoreMesh(core_axis_name="c", subcore_axis_name="s")
_SC_PARAMS = pltpu.CompilerParams(needs_layout_passes=False)


def _dot(a, b):
    return lax.dot(a, b, precision=_HI, preferred_element_type=_f32)


_CGS = 128
_NCS = NP // _CGS
_NBS = 4


def _sc_stats_body(row_flat, col_flat, ea_hbm, zero_n, zero_a,
                   cr_out, cc_out, a_out,
                   ridx, cidx, ea_v, cr_v, cc_v, a_sh, sem_a):
    cid = lax.axis_index("c")
    sid = lax.axis_index("s")
    wid = sid * NC + cid
    base0 = wid * NP
    pltpu.sync_copy(zero_n, cr_v)
    pltpu.sync_copy(zero_n, cc_v)
    pltpu.sync_copy(zero_a.at[pl.ds(sid * RPT, RPT)],
                    a_sh.at[pl.ds(sid * RPT, RPT)])
    plsc.subcore_barrier()
    ones = jnp.ones((16,), _f32)

    def step(k, carry):
        base = base0 + k * _CGS
        d1 = pltpu.async_copy(row_flat.at[pl.ds(base, _CGS)], ridx, sem_a)
        d2 = pltpu.async_copy(col_flat.at[pl.ds(base, _CGS)], cidx, sem_a)
        d3 = pltpu.async_copy(ea_hbm.at[pl.ds(base, _CGS)], ea_v, sem_a)
        d1.wait()
        d2.wait()
        d3.wait()
        for i in range(_CGS // 16):
            sl = pl.ds(i * 16, 16)
            plsc.addupdate_scatter(cr_v, [ridx[sl]], ones)
            plsc.addupdate_scatter(cc_v, [cidx[sl]], ones)
        pltpu.sync_copy(ea_v, a_sh.at[ridx], add=True)
        return carry

    lax.fori_loop(0, _NCS, step, 0)
    plsc.subcore_barrier()
    pltpu.sync_copy(cr_v, cr_out.at[pl.ds(wid * NP, NP)])
    pltpu.sync_copy(cc_v, cc_out.at[pl.ds(wid * NP, NP)])
    pltpu.sync_copy(a_sh.at[pl.ds(sid * RPT, RPT)],
                    a_out.at[cid, pl.ds(sid * RPT, RPT)])


_sc_stats = pl.kernel(
    _sc_stats_body,
    out_type=(
        jax.ShapeDtypeStruct((NW * NP,), _f32),
        jax.ShapeDtypeStruct((NW * NP,), _f32),
        jax.ShapeDtypeStruct((NC, NP, DE), _f32),
    ),
    mesh=_SC_MESH,
    scratch_types=[
        pltpu.VMEM((_CGS,), jnp.int32),
        pltpu.VMEM((_CGS,), jnp.int32),
        pltpu.VMEM((_CGS, DE), _f32),
        pltpu.VMEM((NP,), _f32),
        pltpu.VMEM((NP,), _f32),
        pltpu.VMEM_SHARED((NP, DE), _f32),
        pltpu.SemaphoreType.DMA,
    ],
    compiler_params=_SC_PARAMS,
)


_TB2 = 2048
_NT2 = NP // _TB2


_PK = EP // 8
_EB2 = 8192
_NB2 = _PK // _EB2


def _tc_ea_body(pk_ref, sea_ref, gram_ref, sea_acc, g_acc):
    i = pl.program_id(0)

    @pl.when(i == 0)
    def _init():
        sea_acc[...] = jnp.zeros_like(sea_acc)
        g_acc[...] = jnp.zeros_like(g_acc)

    blk = pk_ref[...]
    sea_acc[...] += jnp.sum(blk, axis=0, keepdims=True)
    g_acc[...] += lax.dot_general(
        blk, blk, (((0,), (0,)), ((), ())),
        precision=_HI, preferred_element_type=_f32)

    @pl.when(i == _NB2 - 1)
    def _finish():
        sea = jnp.zeros((1, DE), _f32)
        gram = jnp.zeros((DE, DE), _f32)
        for s in range(8):
            sea = sea + sea_acc[:, s * DE:(s + 1) * DE]
            gram = gram + g_acc[s * DE:(s + 1) * DE, s * DE:(s + 1) * DE]
        sea_ref[...] = sea
        gram_ref[...] = gram


def _tc_ea(pk):
    return pl.pallas_call(
        _tc_ea_body,
        grid=(_NB2,),
        in_specs=[pl.BlockSpec((_EB2, 8 * DE), lambda i: (i, 0))],
        out_specs=[
            pl.BlockSpec((1, DE), lambda i: (0, 0)),
            pl.BlockSpec((DE, DE), lambda i: (0, 0)),
        ],
        out_shape=[
            jax.ShapeDtypeStruct((1, DE), _f32),
            jax.ShapeDtypeStruct((DE, DE), _f32),
        ],
        scratch_shapes=[
            pltpu.VMEM((1, 8 * DE), _f32),
            pltpu.VMEM((8 * DE, 8 * DE), _f32),
        ],
    )(pk)


def _tc_prep_body(x_ref, crp_ref, ap_ref, sea_ref, gram_ref, w1_ref,
                  b1_ref, g1_ref, be1_ref,
                  yp_ref, w1bp_ref, cvec_ref,
                  cr_scr):
    w1a = w1_ref[:D]
    w1b = w1_ref[D:]
    b1 = b1_ref[...]
    cr_scr[...] = jnp.sum(crp_ref[...], axis=0, keepdims=True)
    sea = sea_ref[...]
    gram = gram_ref[...]

    def _stat_tile(t, carry):
        cry, cry2, cya = carry
        off = pl.multiple_of(t * _TB2, _TB2)
        sl = pl.ds(off, _TB2)
        y_t = _dot(x_ref[sl], w1a)
        cr_t = cr_scr[:, pl.ds(off, _TB2)]
        a_t = ap_ref[0, sl] + ap_ref[1, sl]
        cry = cry + _dot(cr_t, y_t)
        cry2 = cry2 + _dot(cr_t, y_t * y_t)
        cya = cya + jnp.sum(y_t * _dot(a_t, w1b), axis=0, keepdims=True)
        return cry, cry2, cya

    zed = jnp.zeros((1, L), _f32)
    cry, cry2, cya = lax.fori_loop(0, _NT2, _stat_tile, (zed, zed, zed))
    seaw = _dot(sea, w1b)
    quad = jnp.sum(w1b * _dot(gram, w1b), axis=0, keepdims=True)
    s1 = cry + seaw + E * b1
    s2 = (cry2 + 2.0 * cya + 2.0 * b1 * cry + quad
          + 2.0 * b1 * seaw + E * b1 * b1)
    m = s1 / E
    v = s2 / E - m * m
    ghat = g1_ref[...] * lax.rsqrt(v + 1e-5)
    cvec_ref[...] = (b1 - m) * ghat + be1_ref[...]
    w1bp_ref[...] = w1b * ghat

    def _write_tile(t, carry):
        sl = pl.ds(pl.multiple_of(t * _TB2, _TB2), _TB2)
        yp_ref[sl] = _dot(x_ref[sl], w1a) * ghat
        return carry

    lax.fori_loop(0, _NT2, _write_tile, 0)


def _tc_prep(x, crp, ap, sea, gram, w1, b1r, g1r, be1r):
    return pl.pallas_call(
        _tc_prep_body,
        grid=(1,),
        in_specs=[
            pl.BlockSpec((NP, D), lambda i: (0, 0)),
            pl.BlockSpec((NW, NP), lambda i: (0, 0)),
            pl.BlockSpec((NC, NP, DE), lambda i: (0, 0, 0)),
            pl.BlockSpec((1, DE), lambda i: (0, 0)),
            pl.BlockSpec((DE, DE), lambda i: (0, 0)),
            pl.BlockSpec((ENTRY, L), lambda i: (0, 0)),
            pl.BlockSpec((1, L), lambda i: (0, 0)),
            pl.BlockSpec((1, L), lambda i: (0, 0)),
            pl.BlockSpec((1, L), lambda i: (0, 0)),
        ],
        out_specs=[
            pl.BlockSpec((NP, L), lambda i: (0, 0)),
            pl.BlockSpec((DE, L), lambda i: (0, 0)),
            pl.BlockSpec((1, L), lambda i: (0, 0)),
        ],
        out_shape=[
            jax.ShapeDtypeStruct((NP, L), _f32),
            jax.ShapeDtypeStruct((DE, L), _f32),
            jax.ShapeDtypeStruct((1, L), _f32),
        ],
        scratch_shapes=[
            pltpu.VMEM((1, NP), _f32),
        ],
    )(x, crp, ap, sea, gram, w1, b1r, g1r, be1r)


_EB3 = 2048
_NB3 = _PK // _EB3


def _tc_z_body(pk_ref, w_ref, c_ref, z_ref):
    z_ref[...] = lax.dot(pk_ref[...], w_ref[...],
                         preferred_element_type=_f32) + c_ref[...]


def _tc_z(pk, w_big, cvec8):
    return pl.pallas_call(
        _tc_z_body,
        grid=(_NB3,),
        in_specs=[
            pl.BlockSpec((_EB3, 8 * DE), lambda i: (i, 0)),
            pl.BlockSpec((8 * DE, 8 * L), lambda i: (0, 0)),
            pl.BlockSpec((1, 8 * L), lambda i: (0, 0)),
        ],
        out_specs=pl.BlockSpec((_EB3, 8 * L), lambda i: (i, 0)),
        out_shape=jax.ShapeDtypeStruct((_PK, 8 * L), _f32),
    )(pk, w_big, cvec8)


_CG4 = 128
_NCH = NP // _CG4


def _sc_edge_body(row_flat, col_flat, yp_hbm, zp_hbm, zero_acc,
                  accp_out,
                  ridx, cidx, rows, zv, acc_sh, sem_a, sem_g):
    cid = lax.axis_index("c")
    sid = lax.axis_index("s")
    wid = sid * NC + cid
    base0 = wid * NP
    pltpu.sync_copy(zero_acc.at[pl.ds(sid * RPT, RPT)],
                    acc_sh.at[pl.ds(sid * RPT, RPT)])
    plsc.subcore_barrier()

    def step(k, carry):
        base = base0 + k * _CG4
        d1 = pltpu.async_copy(row_flat.at[pl.ds(base, _CG4)], ridx, sem_a)
        d2 = pltpu.async_copy(col_flat.at[pl.ds(base, _CG4)], cidx, sem_a)
        zoff = pl.multiple_of(base0 // 8 + k * (_CG4 // 8), 8)
        d3 = pltpu.async_copy(zp_hbm.at[pl.ds(zoff, _CG4 // 8)], zv, sem_a)
        d1.wait()
        gat = pltpu.async_copy(yp_hbm.at[ridx], rows, sem_g)
        d2.wait()
        d3.wait()
        gat.wait()

        def body(i, c2):
            i8 = i // 8
            r8 = i % 8
            for q in range(L // 16):
                sl = pl.ds(q * 16, 16)
                zsl = pl.ds(r8 * L + q * 16, 16)
                rows[i, sl] = jnp.maximum(rows[i, sl] + zv[i8, zsl], 0.0)
            return c2

        lax.fori_loop(0, _CG4, body, 0)
        pltpu.sync_copy(rows, acc_sh.at[cidx], add=True)
        return carry

    lax.fori_loop(0, _NCH, step, 0)
    plsc.subcore_barrier()
    pltpu.sync_copy(acc_sh.at[pl.ds(sid * RPT, RPT)],
                    accp_out.at[cid, pl.ds(sid * RPT, RPT)])


_sc_edge = pl.kernel(
    _sc_edge_body,
    out_type=jax.ShapeDtypeStruct((NC, NP, L), _f32),
    mesh=_SC_MESH,
    scratch_types=[
        pltpu.VMEM((_CG4,), jnp.int32),
        pltpu.VMEM((_CG4,), jnp.int32),
        pltpu.VMEM((_CG4, L), _f32),
        pltpu.VMEM((_CG4 // 8, 8 * L), _f32),
        pltpu.VMEM_SHARED((NP, L), _f32),
        pltpu.SemaphoreType.DMA,
        pltpu.SemaphoreType.DMA,
    ],
    compiler_params=_SC_PARAMS,
)


_TB5 = 2000
_NT5 = N // _TB5


def _tc_node_body(x_ref, accp_ref, ccpt_ref, w2_ref, b2_ref, g2_ref,
                  be2_ref, w3_ref, b3_ref, out_ref, t_scr):
    w2a = w2_ref[:D]
    w2b = w2_ref[D:]
    s1 = jnp.zeros((1, SECOND), _f32)
    s2 = jnp.zeros((1, SECOND), _f32)
    for t in range(_NT5):
        sl = pl.ds(t * _TB5, _TB5)
        cnt = jnp.sum(ccpt_ref[sl], axis=1, keepdims=True)
        mean = (accp_ref[0, sl] + accp_ref[1, sl]) / jnp.maximum(cnt, 1.0)
        tt = _dot(x_ref[sl], w2a) + _dot(mean, w2b) + b2_ref[...]
        t_scr[sl] = tt
        s1 = s1 + jnp.sum(tt, axis=0, keepdims=True)
        s2 = s2 + jnp.sum(tt * tt, axis=0, keepdims=True)
    m2 = s1 / N
    v2 = s2 / N - m2 * m2
    scale = lax.rsqrt(v2 + 1e-5) * g2_ref[...]
    for t in range(_NT5):
        sl = pl.ds(t * _TB5, _TB5)
        tt = jnp.maximum((t_scr[sl] - m2) * scale + be2_ref[...], 0.0)
        out_ref[sl] = jnp.maximum(_dot(tt, w3_ref[...]) + b3_ref[...], 0.0)


def _tc_node(x, accp, ccpt, w2, b2r, g2r, be2r, w3, b3r):
    return pl.pallas_call(
        _tc_node_body,
        grid=(1,),
        in_specs=[
            pl.BlockSpec((N, D), lambda i: (0, 0)),
            pl.BlockSpec((NC, N, L), lambda i: (0, 0, 0)),
            pl.BlockSpec((N, NW), lambda i: (0, 0)),
            pl.BlockSpec((SECOND, SECOND), lambda i: (0, 0)),
            pl.BlockSpec((1, SECOND), lambda i: (0, 0)),
            pl.BlockSpec((1, SECOND), lambda i: (0, 0)),
            pl.BlockSpec((1, SECOND), lambda i: (0, 0)),
            pl.BlockSpec((SECOND, D), lambda i: (0, 0)),
            pl.BlockSpec((1, D), lambda i: (0, 0)),
        ],
        out_specs=pl.BlockSpec((N, D), lambda i: (0, 0)),
        out_shape=jax.ShapeDtypeStruct((N, D), _f32),
        scratch_shapes=[pltpu.VMEM((N, SECOND), _f32)],
    )(x, accp, ccpt, w2, b2r, g2r, be2r, w3, b3r)


def kernel(x, edge_index, edge_attr, u, batch,
           W1, b1, g1, be1, W2, b2, g2, be2, W3, b3):
    del u, batch
    npad = NP - N
    epad = EP - E
    pad_nodes = N + (jnp.arange(epad, dtype=jnp.int32) % npad)
    row_flat = jnp.concatenate([edge_index[0], pad_nodes])
    col_flat = jnp.concatenate([edge_index[1], pad_nodes])
    ea_pad = jnp.concatenate(
        [edge_attr, jnp.zeros((epad, DE), _f32)], axis=0)
    x_pad = jnp.concatenate([x, jnp.zeros((npad, D), _f32)], axis=0)
    zero_n = jnp.zeros((NP,), _f32)
    zero_a = jnp.zeros((NP, DE), _f32)
    zero_acc = jnp.zeros((NP, L), _f32)
    b1r = b1.reshape(1, L)
    g1r = g1.reshape(1, L)
    be1r = be1.reshape(1, L)
    b2r = b2.reshape(1, SECOND)
    g2r = g2.reshape(1, SECOND)
    be2r = be2.reshape(1, SECOND)
    b3r = b3.reshape(1, D)

    cr_f, cc_f, ap = _sc_stats(row_flat, col_flat, ea_pad, zero_n, zero_a)
    crp = cr_f.reshape(NW, NP)
    pack = ea_pad.reshape(_PK, 8 * DE)
    sea, gram = _tc_ea(pack)
    yp, w1bp, cvec = _tc_prep(x_pad, crp, ap, sea, gram, W1, b1r, g1r, be1r)
    eye8 = jnp.eye(8, dtype=_f32)
    w_big = (eye8[:, None, :, None]
             * w1bp[None, :, None, :]).reshape(8 * DE, 8 * L)
    cvec8 = jnp.tile(cvec, (1, 8))
    zp = _tc_z(pack, w_big, cvec8)
    accp = _sc_edge(row_flat, col_flat, yp, zp, zero_acc)
    ccpt = cc_f.reshape(NW, NP).T
    return _tc_node(x, accp, ccpt, W2, b2r, g2r, be2r, W3, b3r)


# speedup vs baseline: 1.1697x; 1.1697x over previous
import jax
import jax.numpy as jnp
from jax import lax
from jax.experimental import pallas as pl
from jax.experimental.pallas import tpu as pltpu
from jax.experimental.pallas import tpu_sc as plsc

N = 10000
E = 320000
D = 128
DE = 16
L = 128
ENTRY = D + DE
SECOND = D + L

NC = 2
NS = 16
NW = NC * NS
NP = 10240
EP = NW * NP
CG = 80
GPW = NP // CG
RPT = NP // NS

_HI = lax.Precision.HIGHEST
_f32 = jnp.float32

_SC_MESH = plsc.VectorSubcoreMesh(core_axis_name="c", subcore_axis_name="s")
_SC_PARAMS = pltpu.CompilerParams(needs_layout_passes=False)


def _dot(a, b):
    return lax.dot(a, b, precision=_HI, preferred_element_type=_f32)


_CGS = 128
_NCS = NP // _CGS
_NBS = 4


def _sc_stats_body(row_flat, col_flat, ea_hbm, zero_n, zero_a,
                   cr_out, cc_out, a_out,
                   ridx, cidx, ea_v, cr_v, cc_v, a_sh, sem_a):
    cid = lax.axis_index("c")
    sid = lax.axis_index("s")
    wid = sid * NC + cid
    base0 = wid * NP
    pltpu.sync_copy(zero_n, cr_v)
    pltpu.sync_copy(zero_n, cc_v)
    pltpu.sync_copy(zero_a.at[pl.ds(sid * RPT, RPT)],
                    a_sh.at[pl.ds(sid * RPT, RPT)])
    plsc.subcore_barrier()
    ones = jnp.ones((16,), _f32)

    def step(k, carry):
        base = base0 + k * _CGS
        d1 = pltpu.async_copy(row_flat.at[pl.ds(base, _CGS)], ridx, sem_a)
        d2 = pltpu.async_copy(col_flat.at[pl.ds(base, _CGS)], cidx, sem_a)
        d3 = pltpu.async_copy(ea_hbm.at[pl.ds(base, _CGS)], ea_v, sem_a)
        d1.wait()
        d2.wait()
        d3.wait()
        for i in range(_CGS // 16):
            sl = pl.ds(i * 16, 16)
            plsc.addupdate_scatter(cr_v, [ridx[sl]], ones)
            plsc.addupdate_scatter(cc_v, [cidx[sl]], ones)
        pltpu.sync_copy(ea_v, a_sh.at[ridx], add=True)
        return carry

    lax.fori_loop(0, _NCS, step, 0)
    plsc.subcore_barrier()
    pltpu.sync_copy(cr_v, cr_out.at[pl.ds(wid * NP, NP)])
    pltpu.sync_copy(cc_v, cc_out.at[pl.ds(wid * NP, NP)])
    pltpu.sync_copy(a_sh.at[pl.ds(sid * RPT, RPT)],
                    a_out.at[cid, pl.ds(sid * RPT, RPT)])


_sc_stats = pl.kernel(
    _sc_stats_body,
    out_type=(
        jax.ShapeDtypeStruct((NW * NP,), _f32),
        jax.ShapeDtypeStruct((NW * NP,), _f32),
        jax.ShapeDtypeStruct((NC, NP, DE), _f32),
    ),
    mesh=_SC_MESH,
    scratch_types=[
        pltpu.VMEM((_CGS,), jnp.int32),
        pltpu.VMEM((_CGS,), jnp.int32),
        pltpu.VMEM((_CGS, DE), _f32),
        pltpu.VMEM((NP,), _f32),
        pltpu.VMEM((NP,), _f32),
        pltpu.VMEM_SHARED((NP, DE), _f32),
        pltpu.SemaphoreType.DMA,
    ],
    compiler_params=_SC_PARAMS,
)


_TB2 = 2048
_NT2 = NP // _TB2


_PK = EP // 8
_EB2 = 8192
_NB2 = _PK // _EB2


def _tc_ea_body(pk_ref, sea_ref, gram_ref, sea_acc, g_acc):
    i = pl.program_id(0)

    @pl.when(i == 0)
    def _init():
        sea_acc[...] = jnp.zeros_like(sea_acc)
        g_acc[...] = jnp.zeros_like(g_acc)

    blk = pk_ref[...]
    sea_acc[...] += jnp.sum(blk, axis=0, keepdims=True)
    g_acc[...] += lax.dot_general(
        blk, blk, (((0,), (0,)), ((), ())),
        precision=_HI, preferred_element_type=_f32)

    @pl.when(i == _NB2 - 1)
    def _finish():
        sea = jnp.zeros((1, DE), _f32)
        gram = jnp.zeros((DE, DE), _f32)
        for s in range(8):
            sea = sea + sea_acc[:, s * DE:(s + 1) * DE]
            gram = gram + g_acc[s * DE:(s + 1) * DE, s * DE:(s + 1) * DE]
        sea_ref[...] = sea
        gram_ref[...] = gram


def _tc_ea(pk):
    return pl.pallas_call(
        _tc_ea_body,
        grid=(_NB2,),
        in_specs=[pl.BlockSpec((_EB2, 8 * DE), lambda i: (i, 0))],
        out_specs=[
            pl.BlockSpec((1, DE), lambda i: (0, 0)),
            pl.BlockSpec((DE, DE), lambda i: (0, 0)),
        ],
        out_shape=[
            jax.ShapeDtypeStruct((1, DE), _f32),
            jax.ShapeDtypeStruct((DE, DE), _f32),
        ],
        scratch_shapes=[
            pltpu.VMEM((1, 8 * DE), _f32),
            pltpu.VMEM((8 * DE, 8 * DE), _f32),
        ],
    )(pk)


def _tc_prep_body(x_ref, crp_ref, ap_ref, sea_ref, gram_ref, w1_ref,
                  b1_ref, g1_ref, be1_ref,
                  yp_ref, w1bp_ref, cvec_ref,
                  cr_scr):
    w1a = w1_ref[:D]
    w1b = w1_ref[D:]
    b1 = b1_ref[...]
    cr_scr[...] = jnp.sum(crp_ref[...], axis=0, keepdims=True)
    sea = sea_ref[...]
    gram = gram_ref[...]

    def _stat_tile(t, carry):
        cry, cry2, cya = carry
        off = pl.multiple_of(t * _TB2, _TB2)
        sl = pl.ds(off, _TB2)
        y_t = _dot(x_ref[sl], w1a)
        cr_t = cr_scr[:, pl.ds(off, _TB2)]
        a_t = ap_ref[0, sl] + ap_ref[1, sl]
        cry = cry + _dot(cr_t, y_t)
        cry2 = cry2 + _dot(cr_t, y_t * y_t)
        cya = cya + jnp.sum(y_t * _dot(a_t, w1b), axis=0, keepdims=True)
        return cry, cry2, cya

    zed = jnp.zeros((1, L), _f32)
    cry, cry2, cya = lax.fori_loop(0, _NT2, _stat_tile, (zed, zed, zed))
    seaw = _dot(sea, w1b)
    quad = jnp.sum(w1b * _dot(gram, w1b), axis=0, keepdims=True)
    s1 = cry + seaw + E * b1
    s2 = (cry2 + 2.0 * cya + 2.0 * b1 * cry + quad
          + 2.0 * b1 * seaw + E * b1 * b1)
    m = s1 / E
    v = s2 / E - m * m
    ghat = g1_ref[...] * lax.rsqrt(v + 1e-5)
    cvec_ref[...] = (b1 - m) * ghat + be1_ref[...]
    w1bp_ref[...] = w1b * ghat

    def _write_tile(t, carry):
        sl = pl.ds(pl.multiple_of(t * _TB2, _TB2), _TB2)
        yp_ref[sl] = _dot(x_ref[sl], w1a) * ghat
        return carry

    lax.fori_loop(0, _NT2, _write_tile, 0)


def _tc_prep(x, crp, ap, sea, gram, w1, b1r, g1r, be1r):
    return pl.pallas_call(
        _tc_prep_body,
        grid=(1,),
        in_specs=[
            pl.BlockSpec((NP, D), lambda i: (0, 0)),
            pl.BlockSpec((NW, NP), lambda i: (0, 0)),
            pl.BlockSpec((NC, NP, DE), lambda i: (0, 0, 0)),
            pl.BlockSpec((1, DE), lambda i: (0, 0)),
            pl.BlockSpec((DE, DE), lambda i: (0, 0)),
            pl.BlockSpec((ENTRY, L), lambda i: (0, 0)),
            pl.BlockSpec((1, L), lambda i: (0, 0)),
            pl.BlockSpec((1, L), lambda i: (0, 0)),
            pl.BlockSpec((1, L), lambda i: (0, 0)),
        ],
        out_specs=[
            pl.BlockSpec((NP, L), lambda i: (0, 0)),
            pl.BlockSpec((DE, L), lambda i: (0, 0)),
            pl.BlockSpec((1, L), lambda i: (0, 0)),
        ],
        out_shape=[
            jax.ShapeDtypeStruct((NP, L), _f32),
            jax.ShapeDtypeStruct((DE, L), _f32),
            jax.ShapeDtypeStruct((1, L), _f32),
        ],
        scratch_shapes=[
            pltpu.VMEM((1, NP), _f32),
        ],
    )(x, crp, ap, sea, gram, w1, b1r, g1r, be1r)


_EB3 = 2048
_NB3 = _PK // _EB3


def _tc_z_body(pk_ref, w_ref, c_ref, z_ref):
    z_ref[...] = lax.dot(pk_ref[...], w_ref[...],
                         preferred_element_type=_f32) + c_ref[...]


def _tc_z(pk, w_big, cvec8):
    return pl.pallas_call(
        _tc_z_body,
        grid=(_NB3,),
        in_specs=[
            pl.BlockSpec((_EB3, 8 * DE), lambda i: (i, 0)),
            pl.BlockSpec((8 * DE, 8 * L), lambda i: (0, 0)),
            pl.BlockSpec((1, 8 * L), lambda i: (0, 0)),
        ],
        out_specs=pl.BlockSpec((_EB3, 8 * L), lambda i: (i, 0)),
        out_shape=jax.ShapeDtypeStruct((_PK, 8 * L), _f32),
    )(pk, w_big, cvec8)


_CG4 = 128
_NCH = NP // _CG4


def _sc_edge_body(row_flat, col_flat, yp_hbm, zp_hbm, zero_acc,
                  accp_out,
                  ridx, cidx, rows, zv, acc_sh, sem_a, sem_g):
    cid = lax.axis_index("c")
    sid = lax.axis_index("s")
    wid = sid * NC + cid
    base0 = wid * NP
    pltpu.sync_copy(zero_acc.at[pl.ds(sid * RPT, RPT)],
                    acc_sh.at[pl.ds(sid * RPT, RPT)])
    plsc.subcore_barrier()

    def step(k, carry):
        base = base0 + k * _CG4
        d1 = pltpu.async_copy(row_flat.at[pl.ds(base, _CG4)], ridx, sem_a)
        d2 = pltpu.async_copy(col_flat.at[pl.ds(base, _CG4)], cidx, sem_a)
        d3 = pltpu.async_copy(zp_hbm.at[pl.ds(base, _CG4)], zv, sem_a)
        d1.wait()
        gat = pltpu.async_copy(yp_hbm.at[ridx], rows, sem_g)
        d2.wait()
        d3.wait()
        gat.wait()

        def body(i, c2):
            for q in range(L // 16):
                sl = pl.ds(q * 16, 16)
                rows[i, sl] = jnp.maximum(rows[i, sl] + zv[i, sl], 0.0)
            return c2

        lax.fori_loop(0, _CG4, body, 0)
        pltpu.sync_copy(rows, acc_sh.at[cidx], add=True)
        return carry

    lax.fori_loop(0, _NCH, step, 0)
    plsc.subcore_barrier()
    pltpu.sync_copy(acc_sh.at[pl.ds(sid * RPT, RPT)],
                    accp_out.at[cid, pl.ds(sid * RPT, RPT)])


_sc_edge = pl.kernel(
    _sc_edge_body,
    out_type=jax.ShapeDtypeStruct((NC, NP, L), _f32),
    mesh=_SC_MESH,
    scratch_types=[
        pltpu.VMEM((_CG4,), jnp.int32),
        pltpu.VMEM((_CG4,), jnp.int32),
        pltpu.VMEM((_CG4, L), _f32),
        pltpu.VMEM((_CG4, L), _f32),
        pltpu.VMEM_SHARED((NP, L), _f32),
        pltpu.SemaphoreType.DMA,
        pltpu.SemaphoreType.DMA,
    ],
    compiler_params=_SC_PARAMS,
)


_TB5 = 2000
_NT5 = N // _TB5


def _tc_node_body(x_ref, accp_ref, ccpt_ref, w2_ref, b2_ref, g2_ref,
                  be2_ref, w3_ref, b3_ref, out_ref, t_scr):
    w2a = w2_ref[:D]
    w2b = w2_ref[D:]
    s1 = jnp.zeros((1, SECOND), _f32)
    s2 = jnp.zeros((1, SECOND), _f32)
    for t in range(_NT5):
        sl = pl.ds(t * _TB5, _TB5)
        cnt = jnp.sum(ccpt_ref[sl], axis=1, keepdims=True)
        mean = (accp_ref[0, sl] + accp_ref[1, sl]) / jnp.maximum(cnt, 1.0)
        tt = _dot(x_ref[sl], w2a) + _dot(mean, w2b) + b2_ref[...]
        t_scr[sl] = tt
        s1 = s1 + jnp.sum(tt, axis=0, keepdims=True)
        s2 = s2 + jnp.sum(tt * tt, axis=0, keepdims=True)
    m2 = s1 / N
    v2 = s2 / N - m2 * m2
    scale = lax.rsqrt(v2 + 1e-5) * g2_ref[...]
    for t in range(_NT5):
        sl = pl.ds(t * _TB5, _TB5)
        tt = jnp.maximum((t_scr[sl] - m2) * scale + be2_ref[...], 0.0)
        out_ref[sl] = jnp.maximum(_dot(tt, w3_ref[...]) + b3_ref[...], 0.0)


def _tc_node(x, accp, ccpt, w2, b2r, g2r, be2r, w3, b3r):
    return pl.pallas_call(
        _tc_node_body,
        grid=(1,),
        in_specs=[
            pl.BlockSpec((N, D), lambda i: (0, 0)),
            pl.BlockSpec((NC, N, L), lambda i: (0, 0, 0)),
            pl.BlockSpec((N, NW), lambda i: (0, 0)),
            pl.BlockSpec((SECOND, SECOND), lambda i: (0, 0)),
            pl.BlockSpec((1, SECOND), lambda i: (0, 0)),
            pl.BlockSpec((1, SECOND), lambda i: (0, 0)),
            pl.BlockSpec((1, SECOND), lambda i: (0, 0)),
            pl.BlockSpec((SECOND, D), lambda i: (0, 0)),
            pl.BlockSpec((1, D), lambda i: (0, 0)),
        ],
        out_specs=pl.BlockSpec((N, D), lambda i: (0, 0)),
        out_shape=jax.ShapeDtypeStruct((N, D), _f32),
        scratch_shapes=[pltpu.VMEM((N, SECOND), _f32)],
    )(x, accp, ccpt, w2, b2r, g2r, be2r, w3, b3r)


def kernel(x, edge_index, edge_attr, u, batch,
           W1, b1, g1, be1, W2, b2, g2, be2, W3, b3):
    del u, batch
    npad = NP - N
    epad = EP - E
    pad_nodes = N + (jnp.arange(epad, dtype=jnp.int32) % npad)
    row_flat = jnp.concatenate([edge_index[0], pad_nodes])
    col_flat = jnp.concatenate([edge_index[1], pad_nodes])
    ea_pad = jnp.concatenate(
        [edge_attr, jnp.zeros((epad, DE), _f32)], axis=0)
    x_pad = jnp.concatenate([x, jnp.zeros((npad, D), _f32)], axis=0)
    zero_n = jnp.zeros((NP,), _f32)
    zero_a = jnp.zeros((NP, DE), _f32)
    zero_acc = jnp.zeros((NP, L), _f32)
    b1r = b1.reshape(1, L)
    g1r = g1.reshape(1, L)
    be1r = be1.reshape(1, L)
    b2r = b2.reshape(1, SECOND)
    g2r = g2.reshape(1, SECOND)
    be2r = be2.reshape(1, SECOND)
    b3r = b3.reshape(1, D)

    cr_f, cc_f, ap = _sc_stats(row_flat, col_flat, ea_pad, zero_n, zero_a)
    crp = cr_f.reshape(NW, NP)
    pack = ea_pad.reshape(_PK, 8 * DE)
    sea, gram = _tc_ea(pack)
    yp, w1bp, cvec = _tc_prep(x_pad, crp, ap, sea, gram, W1, b1r, g1r, be1r)
    eye8 = jnp.eye(8, dtype=_f32)
    w_big = (eye8[:, None, :, None]
             * w1bp[None, :, None, :]).reshape(8 * DE, 8 * L)
    cvec8 = jnp.tile(cvec, (1, 8))
    zp = _tc_z(pack, w_big, cvec8).reshape(EP, L)
    accp = _sc_edge(row_flat, col_flat, yp, zp, zero_acc)
    ccpt = cc_f.reshape(NW, NP).T
    return _tc_node(x, accp, ccpt, W2, b2r, g2r, be2r, W3, b3r)
